# Initial kernel scaffold; baseline (speedup 1.0000x reference)
import jax
import jax.numpy as jnp
from jax.experimental import pallas as pl


def kernel(_pooling_out, indexes, all_radii, all_angles):
    raise NotImplementedError("write your pallas kernel here")


# speedup vs baseline: 4.5792x; 4.5792x over previous
import functools

import jax
import jax.numpy as jnp
from jax import lax
from jax.experimental import pallas as pl
from jax.experimental.pallas import tpu as pltpu
from jax.experimental.pallas import tpu_sc as plsc
import numpy as np

MIN_RADIUS = 0.5
MAX_RADIUS = 4.0
MIN_ANGLE = 30.0 * np.pi / 180.0
MAX_ANGLE = 180.0 * np.pi / 180.0
MAX_PED = 100

_M = 2_000_000
_B = 500_000

_NS = 16
_ROW = _M // _NS
_COPY_CHUNK = 25_000
_NB = 50
_CB = _B // _NB


def _per_core(s, idx_hbm, in_hbm, out_hbm, const_hbm, copy_v, idx_v, vals_v,
              sem):
    for k in range(_ROW // _COPY_CHUNK):
        base = s * _ROW + k * _COPY_CHUNK
        pltpu.sync_copy(in_hbm.at[pl.ds(base, _COPY_CHUNK)], copy_v)
        pltpu.sync_copy(copy_v, out_hbm.at[pl.ds(base, _COPY_CHUNK)])
    plsc.subcore_barrier()
    pltpu.sync_copy(const_hbm, vals_v)

    def scatter_block(i, carry):
        blk = s + i * _NS
        pltpu.sync_copy(idx_hbm.at[pl.ds(blk * _CB, _CB)], idx_v)
        pltpu.async_copy(vals_v, out_hbm.at[idx_v], sem).wait()
        return carry

    n_local = (_NB - s + _NS - 1) // _NS
    lax.fori_loop(0, n_local, scatter_block, 0)


def _body(idx_hbm, radii_hbm, angles_hbm, cr_hbm, ca_hbm, out_r, out_a,
          copy_v, idx_v, vals_v, sem):
    c = lax.axis_index("c")
    s = lax.axis_index("s")

    @pl.when(c == 0)
    def _():
        _per_core(s, idx_hbm, radii_hbm, out_r, cr_hbm, copy_v, idx_v, vals_v,
                  sem)

    @pl.when(c == 1)
    def _():
        _per_core(s, idx_hbm, angles_hbm, out_a, ca_hbm, copy_v, idx_v,
                  vals_v, sem)


_sc_call = pl.kernel(
    _body,
    out_type=(
        jax.ShapeDtypeStruct((_M,), jnp.float32),
        jax.ShapeDtypeStruct((_M,), jnp.float32),
    ),
    mesh=plsc.VectorSubcoreMesh(core_axis_name="c", subcore_axis_name="s"),
    scratch_types=(
        pltpu.VMEM((_COPY_CHUNK,), jnp.float32),
        pltpu.VMEM((_CB,), jnp.int32),
        pltpu.VMEM((_CB,), jnp.float32),
        pltpu.SemaphoreType.DMA,
    ),
)


@jax.jit
def kernel(_pooling_out, indexes, all_radii, all_angles):
    radii_val = jnp.full((_CB,), MAX_RADIUS, dtype=jnp.float32)
    angle_val = jnp.full((_CB,), MAX_ANGLE, dtype=jnp.float32)
    idx32 = indexes.astype(jnp.int32)
    return _sc_call(idx32, all_radii, all_angles, radii_val, angle_val)
